# baseline (speedup 1.0000x reference)
import functools

import jax
import jax.numpy as jnp
from jax import lax
from jax.experimental import pallas as pl
from jax.experimental.pallas import tpu as pltpu
from jax.experimental.pallas import tpu_sc as plsc

N_NODES = 10000
N_EDGES = 160000
D = 256
L = 16
NS = 16
MBLK = 400
CHUNK = 48
NBUF = 4
K0 = 136
K1 = 76
KMAX = max(K0, K1)
E_PAD = (K0 + K1) * NS * CHUNK
E_IDX_PAD = (NS * K0 + (NS - 1) * K1 + KMAX) * CHUNK


def _mm_body(x_ref, wa_ref, wb_ref, b1_ref, a_ref, b_ref):
    xb = x_ref[...]
    a_ref[...] = (
        jnp.dot(xb, wa_ref[...], preferred_element_type=jnp.float32)
        + b1_ref[...]
    ).astype(jnp.bfloat16)
    b_ref[...] = jnp.dot(
        xb, wb_ref[...], preferred_element_type=jnp.float32
    ).astype(jnp.bfloat16)


def _node_tables(x_pad, w1a, w1b, b1row):
    grid = (N_NODES // MBLK,)
    return pl.pallas_call(
        _mm_body,
        grid=grid,
        in_specs=[
            pl.BlockSpec((MBLK, D), lambda i: (i, 0)),
            pl.BlockSpec((D, D), lambda i: (0, 0)),
            pl.BlockSpec((D, D), lambda i: (0, 0)),
            pl.BlockSpec((1, D), lambda i: (0, 0)),
        ],
        out_specs=[
            pl.BlockSpec((MBLK, D), lambda i: (i, 0)),
            pl.BlockSpec((MBLK, D), lambda i: (i, 0)),
        ],
        out_shape=[
            jax.ShapeDtypeStruct((N_NODES, D), jnp.bfloat16),
            jax.ShapeDtypeStruct((N_NODES, D), jnp.bfloat16),
        ],
    )(x_pad, w1a, w1b, b1row)


def _edge_body(a_hbm, b_hbm, src_hbm, dst_hbm, w2b2_hbm, out_hbm,
               srcall, dstall, outv, w2v, accbuf, *bufs_and_sems):
    avs = bufs_and_sems[0:NBUF]
    bvs = bufs_and_sems[NBUF:2 * NBUF]
    sems = bufs_and_sems[2 * NBUF:3 * NBUF]
    c = lax.axis_index("c")
    s = lax.axis_index("s")
    is0 = c == 0
    nchunk = jnp.where(is0, K0, K1)
    base0 = jnp.where(
        is0, s * (K0 * CHUNK), NS * K0 * CHUNK + s * (K1 * CHUNK))

    pltpu.sync_copy(src_hbm.at[pl.ds(base0, KMAX * CHUNK)], srcall)
    pltpu.sync_copy(dst_hbm.at[pl.ds(base0, KMAX * CHUNK)], dstall)
    pltpu.sync_copy(w2b2_hbm, w2v)
    w2lo = [w2v[pl.ds(2 * j * L, L)] for j in range(D // (2 * L))]
    w2hi = [w2v[pl.ds((2 * j + 1) * L, L)] for j in range(D // (2 * L))]
    b2vec = w2v[pl.ds(D, L)]
    rows = lax.iota(jnp.int32, L)
    himask = jnp.full((L,), -65536, jnp.int32)

    def start(k, b):
        off = pl.ds(k * CHUNK, CHUNK)
        pltpu.async_copy(a_hbm.at[srcall.at[off]], avs[b], sems[b])
        pltpu.async_copy(b_hbm.at[dstall.at[off]], bvs[b], sems[b])

    def wait(k, b):
        off = pl.ds(k * CHUNK, CHUNK)
        pltpu.make_async_copy(a_hbm.at[srcall.at[off]], avs[b], sems[b]).wait()
        pltpu.make_async_copy(b_hbm.at[dstall.at[off]], bvs[b], sems[b]).wait()

    for b in range(NBUF - 1):
        start(b, b)

    def outer_body(kk, _):
        for b in range(NBUF):
            k = kk * NBUF + b
            av, bv = avs[b], bvs[b]

            @pl.when(k + NBUF - 1 < nchunk)
            def _():
                start(k + NBUF - 1, (b + NBUF - 1) % NBUF)

            wait(k, b)

            def group_body(g, _):
                def edge_body(i, _):
                    e = g * L + i
                    acc = b2vec
                    for j in range(D // (2 * L)):
                        a32 = plsc.bitcast(
                            av[e, pl.ds(j * L, L)], jnp.bfloat16)
                        b32 = plsc.bitcast(
                            bv[e, pl.ds(j * L, L)], jnp.bfloat16)
                        h = jnp.maximum(a32 + b32, jnp.bfloat16(0))
                        hbits = plsc.bitcast(h, jnp.int32)
                        h_lo = plsc.bitcast(hbits << 16, jnp.float32)
                        h_hi = plsc.bitcast(hbits & himask, jnp.float32)
                        acc = acc + h_lo * w2lo[j] + h_hi * w2hi[j]
                    accbuf[pl.ds(i * L, L)] = acc
                    return 0

                lax.fori_loop(0, L, edge_body, 0)
                out16 = plsc.load_gather(accbuf, [rows * L])
                for j in range(1, L):
                    out16 = out16 + plsc.load_gather(accbuf, [rows * L + j])
                outv[pl.ds(g * L, L)] = out16
                return 0

            lax.fori_loop(0, CHUNK // L, group_body, 0)
            pltpu.sync_copy(outv, out_hbm.at[pl.ds(base0 + k * CHUNK, CHUNK)])
        return 0

    lax.fori_loop(0, nchunk // NBUF, outer_body, 0)


def _edge_scores(a_tab, b_tab, src_idx, dst_idx, w2b2):
    mesh = plsc.VectorSubcoreMesh(core_axis_name="c", subcore_axis_name="s")
    f = pl.kernel(
        _edge_body,
        out_type=jax.ShapeDtypeStruct((E_PAD,), jnp.float32),
        mesh=mesh,
        compiler_params=pltpu.CompilerParams(needs_layout_passes=False),
        scratch_types=(
            [
                pltpu.VMEM((KMAX * CHUNK,), jnp.int32),
                pltpu.VMEM((KMAX * CHUNK,), jnp.int32),
                pltpu.VMEM((CHUNK,), jnp.float32),
                pltpu.VMEM((D + L,), jnp.float32),
                pltpu.VMEM((L * L,), jnp.float32),
            ]
            + [pltpu.VMEM((CHUNK, D // 2), jnp.int32)] * (2 * NBUF)
            + [pltpu.SemaphoreType.DMA] * NBUF
        ),
    )
    return f(a_tab, b_tab, src_idx, dst_idx, w2b2)


@jax.jit
def kernel(x, edge_index, W1, b1, W2, b2):
    w1a = W1[:D, :]
    w1b = W1[D:, :]
    b1row = b1.reshape(1, D)

    a_tab, b_tab = _node_tables(x, w1a, w1b, b1row)
    a_tab = lax.bitcast_convert_type(
        a_tab.reshape(N_NODES, D // 2, 2), jnp.int32)
    b_tab = lax.bitcast_convert_type(
        b_tab.reshape(N_NODES, D // 2, 2), jnp.int32)

    ei = edge_index.astype(jnp.int32)
    src_idx = jnp.pad(ei[0], (0, E_IDX_PAD - N_EDGES))
    dst_idx = jnp.pad(ei[1], (0, E_IDX_PAD - N_EDGES))
    w2r = W2[:, 0].reshape(D // (2 * L), L, 2)
    w2p = jnp.concatenate(
        [w2r[:, :, 0], w2r[:, :, 1]], axis=1).reshape(D)
    w2b2 = jnp.concatenate([w2p, b2, jnp.zeros((L - 1,), jnp.float32)])

    scores = _edge_scores(a_tab, b_tab, src_idx, dst_idx, w2b2)
    return scores[:N_EDGES]


# speedup vs baseline: 1.7769x; 1.7769x over previous
import functools

import jax
import jax.numpy as jnp
from jax import lax
from jax.experimental import pallas as pl
from jax.experimental.pallas import tpu as pltpu
from jax.experimental.pallas import tpu_sc as plsc

N_NODES = 10000
N_EDGES = 160000
D = 256
L = 16
NS = 16
MBLK = 400
CHUNK = 48
NBUF = 4
K0 = 136
K1 = 76
KMAX = max(K0, K1)
E_PAD = (K0 + K1) * NS * CHUNK
E_IDX_PAD = (NS * K0 + (NS - 1) * K1 + KMAX) * CHUNK


def _pack_bf16(v):
    vb = v.astype(jnp.bfloat16)
    lo = lax.bitcast_convert_type(vb[:, : D // 2], jnp.uint16)
    hi = lax.bitcast_convert_type(vb[:, D // 2:], jnp.uint16)
    return lo.astype(jnp.int32) | (hi.astype(jnp.int32) << 16)


def _mm_body(x_ref, wa_ref, wb_ref, b1_ref, a_ref, b_ref):
    xb = x_ref[...]
    a_ref[...] = _pack_bf16(
        jnp.dot(xb, wa_ref[...], preferred_element_type=jnp.float32)
        + b1_ref[...])
    b_ref[...] = _pack_bf16(
        jnp.dot(xb, wb_ref[...], preferred_element_type=jnp.float32))


def _node_tables(x_pad, w1a, w1b, b1row):
    grid = (N_NODES // MBLK,)
    return pl.pallas_call(
        _mm_body,
        grid=grid,
        in_specs=[
            pl.BlockSpec((MBLK, D), lambda i: (i, 0)),
            pl.BlockSpec((D, D), lambda i: (0, 0)),
            pl.BlockSpec((D, D), lambda i: (0, 0)),
            pl.BlockSpec((1, D), lambda i: (0, 0)),
        ],
        out_specs=[
            pl.BlockSpec((MBLK, D // 2), lambda i: (i, 0)),
            pl.BlockSpec((MBLK, D // 2), lambda i: (i, 0)),
        ],
        out_shape=[
            jax.ShapeDtypeStruct((N_NODES, D // 2), jnp.int32),
            jax.ShapeDtypeStruct((N_NODES, D // 2), jnp.int32),
        ],
    )(x_pad, w1a, w1b, b1row)


def _edge_body(a_hbm, b_hbm, src_hbm, dst_hbm, w2b2_hbm, out_hbm,
               srcall, dstall, outv, w2v, accbuf, *bufs_and_sems):
    avs = bufs_and_sems[0:NBUF]
    bvs = bufs_and_sems[NBUF:2 * NBUF]
    sems = bufs_and_sems[2 * NBUF:3 * NBUF]
    c = lax.axis_index("c")
    s = lax.axis_index("s")
    is0 = c == 0
    nchunk = jnp.where(is0, K0, K1)
    base0 = jnp.where(
        is0, s * (K0 * CHUNK), NS * K0 * CHUNK + s * (K1 * CHUNK))

    pltpu.sync_copy(src_hbm.at[pl.ds(base0, KMAX * CHUNK)], srcall)
    pltpu.sync_copy(dst_hbm.at[pl.ds(base0, KMAX * CHUNK)], dstall)
    pltpu.sync_copy(w2b2_hbm, w2v)
    w2lo = [w2v[pl.ds(j * L, L)] for j in range(D // (2 * L))]
    w2hi = [w2v[pl.ds(D // 2 + j * L, L)] for j in range(D // (2 * L))]
    b2vec = w2v[pl.ds(D, L)]
    rows = lax.iota(jnp.int32, L)
    himask = jnp.full((L,), -65536, jnp.int32)

    def start(k, b):
        off = pl.ds(k * CHUNK, CHUNK)
        pltpu.async_copy(a_hbm.at[srcall.at[off]], avs[b], sems[b])
        pltpu.async_copy(b_hbm.at[dstall.at[off]], bvs[b], sems[b])

    def wait(k, b):
        off = pl.ds(k * CHUNK, CHUNK)
        pltpu.make_async_copy(a_hbm.at[srcall.at[off]], avs[b], sems[b]).wait()
        pltpu.make_async_copy(b_hbm.at[dstall.at[off]], bvs[b], sems[b]).wait()

    for b in range(NBUF - 1):
        start(b, b)

    def outer_body(kk, _):
        for b in range(NBUF):
            k = kk * NBUF + b
            av, bv = avs[b], bvs[b]

            @pl.when(k + NBUF - 1 < nchunk)
            def _():
                start(k + NBUF - 1, (b + NBUF - 1) % NBUF)

            wait(k, b)

            def group_body(g, _):
                def edge_body(i, _):
                    e = g * L + i
                    acc = b2vec
                    for j in range(D // (2 * L)):
                        a32 = plsc.bitcast(
                            av[e, pl.ds(j * L, L)], jnp.bfloat16)
                        b32 = plsc.bitcast(
                            bv[e, pl.ds(j * L, L)], jnp.bfloat16)
                        h = jnp.maximum(a32 + b32, jnp.bfloat16(0))
                        hbits = plsc.bitcast(h, jnp.int32)
                        h_lo = plsc.bitcast(hbits << 16, jnp.float32)
                        h_hi = plsc.bitcast(hbits & himask, jnp.float32)
                        acc = acc + h_lo * w2lo[j] + h_hi * w2hi[j]
                    accbuf[pl.ds(i * L, L)] = acc
                    return 0

                lax.fori_loop(0, L, edge_body, 0)
                out16 = plsc.load_gather(accbuf, [rows * L])
                for j in range(1, L):
                    out16 = out16 + plsc.load_gather(accbuf, [rows * L + j])
                outv[pl.ds(g * L, L)] = out16
                return 0

            lax.fori_loop(0, CHUNK // L, group_body, 0)
            pltpu.sync_copy(outv, out_hbm.at[pl.ds(base0 + k * CHUNK, CHUNK)])
        return 0

    lax.fori_loop(0, nchunk // NBUF, outer_body, 0)


def _edge_scores(a_tab, b_tab, src_idx, dst_idx, w2b2):
    mesh = plsc.VectorSubcoreMesh(core_axis_name="c", subcore_axis_name="s")
    f = pl.kernel(
        _edge_body,
        out_type=jax.ShapeDtypeStruct((E_PAD,), jnp.float32),
        mesh=mesh,
        compiler_params=pltpu.CompilerParams(needs_layout_passes=False),
        scratch_types=(
            [
                pltpu.VMEM((KMAX * CHUNK,), jnp.int32),
                pltpu.VMEM((KMAX * CHUNK,), jnp.int32),
                pltpu.VMEM((CHUNK,), jnp.float32),
                pltpu.VMEM((D + L,), jnp.float32),
                pltpu.VMEM((L * L,), jnp.float32),
            ]
            + [pltpu.VMEM((CHUNK, D // 2), jnp.int32)] * (2 * NBUF)
            + [pltpu.SemaphoreType.DMA] * NBUF
        ),
    )
    return f(a_tab, b_tab, src_idx, dst_idx, w2b2)


@jax.jit
def kernel(x, edge_index, W1, b1, W2, b2):
    w1a = W1[:D, :]
    w1b = W1[D:, :]
    b1row = b1.reshape(1, D)

    a_tab, b_tab = _node_tables(x, w1a, w1b, b1row)

    ei = edge_index.astype(jnp.int32)
    src_idx = jnp.pad(ei[0], (0, E_IDX_PAD - N_EDGES))
    dst_idx = jnp.pad(ei[1], (0, E_IDX_PAD - N_EDGES))
    w2b2 = jnp.concatenate(
        [W2[:, 0], b2, jnp.zeros((L - 1,), jnp.float32)])

    scores = _edge_scores(a_tab, b_tab, src_idx, dst_idx, w2b2)
    return scores[:N_EDGES]
